# Initial kernel scaffold; baseline (speedup 1.0000x reference)
import jax
import jax.numpy as jnp
from jax.experimental import pallas as pl


def kernel(user_ui_emb, item_ui_emb, uu_emb, ii_emb, ui_values, ii_values, uu_values, ui_edge_index, ii_edge_index, uu_edge_index):
    raise NotImplementedError("write your pallas kernel here")


# speedup vs baseline: 1.7920x; 1.7920x over previous
import functools

import jax
import jax.numpy as jnp
from jax import lax
from jax.experimental import pallas as pl
from jax.experimental.pallas import tpu as pltpu
from jax.experimental.pallas import tpu_sc as plsc

N_USERS = 10000
N_ITEMS = 10000
D = 128
NC = 2
NS = 16
B = 128
EDGE_ALIGN = NS * B
WROWS = 200


def _pad_edges(rows, cols, vals):
    e = rows.shape[0]
    ep = ((e + EDGE_ALIGN - 1) // EDGE_ALIGN) * EDGE_ALIGN
    if ep != e:
        pad = ep - e
        zi = jnp.zeros((pad,), jnp.int32)
        rows = jnp.concatenate([rows, zi])
        cols = jnp.concatenate([cols, zi])
        vals = jnp.concatenate([vals, jnp.zeros((pad,), jnp.float32)])
    return rows, cols, vals


@functools.partial(jax.jit, static_argnums=(4,))
def _spmm(x, rows, cols, vals, n_out):
    h = n_out // NC
    e_pad = rows.shape[0]
    ept = e_pad // NS
    nbat = ept // B
    n_chunks = h // WROWS
    kmax = (n_chunks + NS - 1) // NS
    mesh = plsc.VectorSubcoreMesh(core_axis_name="c", subcore_axis_name="s")

    @functools.partial(
        pl.kernel,
        out_type=jax.ShapeDtypeStruct((n_out, D), jnp.float32),
        mesh=mesh,
        scratch_types=[
            pltpu.VMEM_SHARED((h, D), jnp.float32),
            pltpu.VMEM((WROWS, D), jnp.float32),
            pltpu.VMEM((B,), jnp.int32),
            pltpu.VMEM((B,), jnp.int32),
            pltpu.VMEM((B,), jnp.int32),
            pltpu.VMEM((B,), jnp.float32),
            pltpu.VMEM((B, D), jnp.float32),
            pltpu.SemaphoreType.DMA,
        ],
    )
    def spmm_kernel(x_hbm, rows_hbm, cols_hbm, vals_hbm, out_hbm,
                    acc, zbuf, row_v, col_v, lrow_v, val_v, gat_v, gsem):
        cid = lax.axis_index("c")
        sid = lax.axis_index("s")
        base_row = cid * h

        zero16 = jnp.zeros((16,), jnp.float32)

        def zrow(i, _):
            for k in range(D // 16):
                zbuf[i, pl.ds(k * 16, 16)] = zero16
            return 0
        lax.fori_loop(0, WROWS, zrow, 0)

        def zchunk(k, _):
            c = k * NS + sid

            @pl.when(c < n_chunks)
            def _():
                pltpu.sync_copy(zbuf, acc.at[pl.ds(c * WROWS, WROWS)])
            return 0
        lax.fori_loop(0, kmax, zchunk, 0)
        plsc.subcore_barrier()

        ebase = sid * ept

        def batch(b, _):
            off = ebase + b * B
            pltpu.sync_copy(rows_hbm.at[pl.ds(off, B)], row_v)
            pltpu.sync_copy(cols_hbm.at[pl.ds(off, B)], col_v)
            pltpu.sync_copy(vals_hbm.at[pl.ds(off, B)], val_v)
            pltpu.async_copy(x_hbm.at[col_v], gat_v, gsem).wait()

            def grp(g, _):
                o = g * 16
                r16 = row_v[pl.ds(o, 16)]
                v16 = val_v[pl.ds(o, 16)]
                loc = r16 - base_row
                ok = (loc >= 0) & (loc < h)
                mval = jnp.where(ok, v16, 0.0)
                lrow_v[pl.ds(o, 16)] = jnp.where(ok, loc, 0)
                for j in range(16):
                    s = mval[j]
                    r = o + j
                    for k in range(D // 16):
                        t = gat_v[r, pl.ds(k * 16, 16)]
                        gat_v[r, pl.ds(k * 16, 16)] = t * s
                return 0
            lax.fori_loop(0, B // 16, grp, 0)
            pltpu.sync_copy(gat_v, acc.at[lrow_v], add=True)
            return 0
        lax.fori_loop(0, nbat, batch, 0)
        plsc.subcore_barrier()

        def wchunk(k, _):
            c = k * NS + sid

            @pl.when(c < n_chunks)
            def _():
                pltpu.sync_copy(acc.at[pl.ds(c * WROWS, WROWS)],
                                out_hbm.at[pl.ds(base_row + c * WROWS, WROWS)])
            return 0
        lax.fori_loop(0, kmax, wchunk, 0)

    return spmm_kernel(x, rows, cols, vals)


def _fuse_body(ego_u, ego_i, e1_u, e1_i, e2_u, e2_i, uu_b, ii_b, u_out, i_out):
    third = jnp.float32(1.0 / 3.0)
    mu = (ego_u[...] + e1_u[...] + e2_u[...]) * third
    mi = (ego_i[...] + e1_i[...] + e2_i[...]) * third
    un = uu_b[...]
    inrm_u = un / jnp.maximum(jnp.sqrt(jnp.sum(un * un, axis=1, keepdims=True)), 1e-12)
    iv = ii_b[...]
    inrm_i = iv / jnp.maximum(jnp.sqrt(jnp.sum(iv * iv, axis=1, keepdims=True)), 1e-12)
    u_out[...] = mu + inrm_u
    i_out[...] = mi + inrm_i


@jax.jit
def _fuse(ego, e1, e2, uu, ii):
    rb = 1000
    grid = N_USERS // rb
    bs_u = pl.BlockSpec((rb, D), lambda i: (i, 0))
    bs_i = pl.BlockSpec((rb, D), lambda i: (i + N_USERS // rb, 0))
    return pl.pallas_call(
        _fuse_body,
        grid=(grid,),
        in_specs=[bs_u, bs_i, bs_u, bs_i, bs_u, bs_i, bs_u, bs_u],
        out_specs=[bs_u, bs_u],
        out_shape=[jax.ShapeDtypeStruct((N_USERS, D), jnp.float32),
                   jax.ShapeDtypeStruct((N_ITEMS, D), jnp.float32)],
    )(ego, ego, e1, e1, e2, e2, uu, ii)


def kernel(user_ui_emb, item_ui_emb, uu_emb, ii_emb, ui_values, ii_values,
           uu_values, ui_edge_index, ii_edge_index, uu_edge_index):
    ri, ci, vi = _pad_edges(ii_edge_index[0], ii_edge_index[1], ii_values)
    ru, cu, vu = _pad_edges(uu_edge_index[0], uu_edge_index[1], uu_values)
    rg, cg, vg = _pad_edges(ui_edge_index[0], ui_edge_index[1], ui_values)

    ii = _spmm(ii_emb, ri, ci, vi, N_ITEMS)
    uu = _spmm(uu_emb, ru, cu, vu, N_USERS)

    ego = jnp.concatenate([user_ui_emb, item_ui_emb], axis=0)
    e1 = _spmm(ego, rg, cg, vg, N_USERS + N_ITEMS)
    e2 = _spmm(e1, rg, cg, vg, N_USERS + N_ITEMS)

    u_ui, i_ui = _fuse(ego, e1, e2, uu, ii)
    return (u_ui, i_ui, ii, uu)


# speedup vs baseline: 2.0569x; 1.1478x over previous
import functools

import jax
import jax.numpy as jnp
from jax import lax
from jax.experimental import pallas as pl
from jax.experimental.pallas import tpu as pltpu
from jax.experimental.pallas import tpu_sc as plsc

N_USERS = 10000
N_ITEMS = 10000
D = 128
DH = D // 2
NC = 2
NS = 16
B = 128
EDGE_ALIGN = NS * B * 2
WROWS = 400


def _pad_edges(rows, cols, vals):
    e = rows.shape[0]
    ep = ((e + EDGE_ALIGN - 1) // EDGE_ALIGN) * EDGE_ALIGN
    if ep != e:
        pad = ep - e
        zi = jnp.zeros((pad,), jnp.int32)
        rows = jnp.concatenate([rows, zi])
        cols = jnp.concatenate([cols, zi])
        vals = jnp.concatenate([vals, jnp.zeros((pad,), jnp.float32)])
    return rows, cols, vals


def _stack_halves(*mats):
    return jnp.concatenate([m[:, :DH] for m in mats] +
                           [m[:, DH:] for m in mats], axis=0)


def _unstack(x2, n_out):
    return jnp.concatenate([x2[:n_out], x2[n_out:]], axis=1)


@functools.partial(jax.jit, static_argnums=(4,))
def _spmm(x2, rows, cols2, vals, n_out):
    e_pad = rows.shape[0]
    ept = e_pad // NS
    nbat = ept // B
    n_chunks = n_out // WROWS
    kmax = (n_chunks + NS - 1) // NS
    mesh = plsc.VectorSubcoreMesh(core_axis_name="c", subcore_axis_name="s")

    @functools.partial(
        pl.kernel,
        out_type=jax.ShapeDtypeStruct((NC * n_out, DH), jnp.float32),
        mesh=mesh,
        scratch_types=[
            pltpu.VMEM_SHARED((n_out, DH), jnp.float32),
            pltpu.VMEM((WROWS, DH), jnp.float32),
            pltpu.VMEM((B,), jnp.int32),
            pltpu.VMEM((B,), jnp.int32),
            pltpu.VMEM((B,), jnp.float32),
            pltpu.VMEM((B, DH), jnp.float32),
            pltpu.VMEM((B,), jnp.int32),
            pltpu.VMEM((B,), jnp.int32),
            pltpu.VMEM((B,), jnp.float32),
            pltpu.VMEM((B, DH), jnp.float32),
            pltpu.SemaphoreType.DMA,
            pltpu.SemaphoreType.DMA,
        ],
        compiler_params=pltpu.CompilerParams(use_tc_tiling_on_sc=False),
    )
    def spmm_kernel(x2_hbm, rows_hbm, cols2_hbm, vals_hbm, out_hbm,
                    acc, zbuf, rowA, colA, valA, gatA,
                    rowB, colB, valB, gatB, semA, semB):
        cid = lax.axis_index("c")
        sid = lax.axis_index("s")

        zero16 = jnp.zeros((16,), jnp.float32)

        def zrow(i, _):
            for k in range(DH // 16):
                zbuf[i, pl.ds(k * 16, 16)] = zero16
            return 0
        lax.fori_loop(0, WROWS, zrow, 0)

        def zchunk(k, _):
            c = k * NS + sid

            @pl.when(c < n_chunks)
            def _():
                pltpu.sync_copy(zbuf, acc.at[pl.ds(c * WROWS, WROWS)])
            return 0
        lax.fori_loop(0, kmax, zchunk, 0)
        plsc.subcore_barrier()

        ebase = sid * ept
        cbase = cid * e_pad + ebase

        def load_and_gather(b, row_v, col_v, val_v, gat_v, sem):
            off = ebase + b * B
            pltpu.sync_copy(rows_hbm.at[pl.ds(off, B)], row_v)
            pltpu.sync_copy(cols2_hbm.at[pl.ds(cbase + b * B, B)], col_v)
            pltpu.sync_copy(vals_hbm.at[pl.ds(off, B)], val_v)
            pltpu.async_copy(x2_hbm.at[col_v], gat_v, sem)

        def wait_gather(col_v, gat_v, sem):
            pltpu.make_async_copy(x2_hbm.at[col_v], gat_v, sem).wait()

        def scale_scatter(row_v, val_v, gat_v):
            def grp(g, _):
                o = g * 16
                v16 = val_v[pl.ds(o, 16)]
                for j in range(16):
                    s = v16[j]
                    r = o + j
                    for k in range(DH // 16):
                        t = gat_v[r, pl.ds(k * 16, 16)]
                        gat_v[r, pl.ds(k * 16, 16)] = t * s
                return 0
            lax.fori_loop(0, B // 16, grp, 0)
            pltpu.sync_copy(gat_v, acc.at[row_v], add=True)

        load_and_gather(0, rowA, colA, valA, gatA, semA)

        def pair(i, _):
            b0 = 2 * i
            load_and_gather(b0 + 1, rowB, colB, valB, gatB, semB)
            wait_gather(colA, gatA, semA)
            scale_scatter(rowA, valA, gatA)

            @pl.when(b0 + 2 < nbat)
            def _():
                load_and_gather(b0 + 2, rowA, colA, valA, gatA, semA)
            wait_gather(colB, gatB, semB)
            scale_scatter(rowB, valB, gatB)
            return 0
        lax.fori_loop(0, nbat // 2, pair, 0)
        plsc.subcore_barrier()

        out_off = cid * n_out

        def wchunk(k, _):
            c = k * NS + sid

            @pl.when(c < n_chunks)
            def _():
                pltpu.sync_copy(acc.at[pl.ds(c * WROWS, WROWS)],
                                out_hbm.at[pl.ds(out_off + c * WROWS, WROWS)])
            return 0
        lax.fori_loop(0, kmax, wchunk, 0)

    return spmm_kernel(x2, rows, cols2, vals)


def _fuse_body(el_u, el_i, er_u, er_i, f1l_u, f1l_i, f1r_u, f1r_i,
               f2l_u, f2l_i, f2r_u, f2r_i, uu_b, ii_b, u_out, i_out):
    third = jnp.float32(1.0 / 3.0)
    mu = jnp.concatenate([(el_u[...] + f1l_u[...] + f2l_u[...]) * third,
                          (er_u[...] + f1r_u[...] + f2r_u[...]) * third], axis=1)
    mi = jnp.concatenate([(el_i[...] + f1l_i[...] + f2l_i[...]) * third,
                          (er_i[...] + f1r_i[...] + f2r_i[...]) * third], axis=1)
    un = uu_b[...]
    iv = ii_b[...]
    un = un / jnp.maximum(jnp.sqrt(jnp.sum(un * un, axis=1, keepdims=True)), 1e-12)
    iv = iv / jnp.maximum(jnp.sqrt(jnp.sum(iv * iv, axis=1, keepdims=True)), 1e-12)
    u_out[...] = mu + un
    i_out[...] = mi + iv


@jax.jit
def _fuse(e2s, f1s, f2s, uu, ii):
    rb = 1000
    nb = N_USERS // rb
    grid = nb
    bs_ul = pl.BlockSpec((rb, DH), lambda i: (i, 0))
    bs_il = pl.BlockSpec((rb, DH), lambda i: (i + nb, 0))
    bs_ur = pl.BlockSpec((rb, DH), lambda i: (i + 2 * nb, 0))
    bs_ir = pl.BlockSpec((rb, DH), lambda i: (i + 3 * nb, 0))
    bs_f = pl.BlockSpec((rb, D), lambda i: (i, 0))
    return pl.pallas_call(
        _fuse_body,
        grid=(grid,),
        in_specs=[bs_ul, bs_il, bs_ur, bs_ir,
                  bs_ul, bs_il, bs_ur, bs_ir,
                  bs_ul, bs_il, bs_ur, bs_ir,
                  bs_f, bs_f],
        out_specs=[bs_f, bs_f],
        out_shape=[jax.ShapeDtypeStruct((N_USERS, D), jnp.float32),
                   jax.ShapeDtypeStruct((N_ITEMS, D), jnp.float32)],
    )(e2s, e2s, e2s, e2s, f1s, f1s, f1s, f1s, f2s, f2s, f2s, f2s, uu, ii)


def kernel(user_ui_emb, item_ui_emb, uu_emb, ii_emb, ui_values, ii_values,
           uu_values, ui_edge_index, ii_edge_index, uu_edge_index):
    hg_rows = jnp.concatenate([ii_edge_index[0], uu_edge_index[0] + N_ITEMS])
    hg_cols = jnp.concatenate([ii_edge_index[1], uu_edge_index[1] + N_ITEMS])
    hg_vals = jnp.concatenate([ii_values, uu_values])
    hg_rows, hg_cols, hg_vals = _pad_edges(hg_rows, hg_cols, hg_vals)
    NT = N_ITEMS + N_USERS
    hg_cols2 = jnp.concatenate([hg_cols, hg_cols + NT])
    hg_x2 = _stack_halves(ii_emb, uu_emb)
    hg_out = _spmm(hg_x2, hg_rows, hg_cols2, hg_vals, NT)
    ii = jnp.concatenate([hg_out[:N_ITEMS], hg_out[NT:NT + N_ITEMS]], axis=1)
    uu = jnp.concatenate([hg_out[N_ITEMS:NT], hg_out[NT + N_ITEMS:]], axis=1)

    rg, cg, vg = _pad_edges(ui_edge_index[0], ui_edge_index[1], ui_values)
    cg2 = jnp.concatenate([cg, cg + NT])
    e0s = _stack_halves(user_ui_emb, item_ui_emb)
    f1s = _spmm(e0s, rg, cg2, vg, NT)
    f2s = _spmm(f1s, rg, cg2, vg, NT)

    u_ui, i_ui = _fuse(e0s, f1s, f2s, uu, ii)
    return (u_ui, i_ui, ii, uu)
